# Initial kernel scaffold; baseline (speedup 1.0000x reference)
import jax
import jax.numpy as jnp
from jax.experimental import pallas as pl


def kernel(x, edge_index, edge_attr, W1, b1, W2, b2, W3, b3, Wout, bout):
    raise NotImplementedError("write your pallas kernel here")


# speedup vs baseline: 3.7118x; 3.7118x over previous
import functools

import jax
import jax.numpy as jnp
from jax import lax
from jax.experimental import pallas as pl
from jax.experimental.pallas import tpu as pltpu
from jax.experimental.pallas import tpu_sc as plsc

N = 10000
E = 320000
D = 128

NC = 2
NS = 16
L = 16

C = 128
EC = E // NC
CHUNKS = EC // C
KMAX = -(-CHUNKS // NS)

NZ_FULL = N // C
NTAIL = N - NZ_FULL * C

_MESH = plsc.VectorSubcoreMesh(core_axis_name="c", subcore_axis_name="s")


def _sc_layer_body(h_hbm, w_hbm, src_hbm, dst_hbm, zeros_hbm,
                   acc_out, src_v, dst_v, w_v, g_v, acc_sh, sem_w, sem_g):
    cid = lax.axis_index("c")
    sid = lax.axis_index("s")

    for kz in range(-(-NZ_FULL // NS)):
        zc = sid + NS * kz

        @pl.when(zc < NZ_FULL)
        def _():
            pltpu.sync_copy(zeros_hbm, acc_sh.at[pl.ds(zc * C, C)])

    @pl.when(sid == 0)
    def _():
        pltpu.sync_copy(zeros_hbm.at[pl.ds(0, NTAIL)],
                        acc_sh.at[pl.ds(NZ_FULL * C, NTAIL)])

    plsc.subcore_barrier()

    @pl.loop(0, KMAX)
    def _(k):
        ci = sid + NS * k

        @pl.when(ci < CHUNKS)
        def _():
            cig = cid * CHUNKS + ci
            cw = pltpu.async_copy(w_hbm.at[pl.ds(cig * C, C)], w_v, sem_w)
            pltpu.sync_copy(src_hbm.at[cig], src_v)
            pltpu.sync_copy(dst_hbm.at[cig], dst_v)
            pltpu.async_copy(h_hbm.at[src_v], g_v, sem_g).wait()
            cw.wait()

            @pl.loop(0, C)
            def _(r):
                for cc in range(D // L):
                    sl = pl.ds(cc * L, L)
                    g_v[r, sl] = g_v[r, sl] * w_v[r, sl]

            pltpu.sync_copy(g_v, acc_sh.at[dst_v], add=True)

    plsc.subcore_barrier()

    for kz in range(-(-NZ_FULL // NS)):
        zc = sid + NS * kz

        @pl.when(zc < NZ_FULL)
        def _():
            pltpu.sync_copy(acc_sh.at[pl.ds(zc * C, C)],
                            acc_out.at[pl.ds(cid * N + zc * C, C)])

    @pl.when(sid == 0)
    def _():
        pltpu.sync_copy(acc_sh.at[pl.ds(NZ_FULL * C, NTAIL)],
                        acc_out.at[pl.ds(cid * N + NZ_FULL * C, NTAIL)])


_sc_layer = pl.kernel(
    _sc_layer_body,
    out_type=jax.ShapeDtypeStruct((NC * N, D), jnp.float32),
    mesh=_MESH,
    scratch_types=[
        pltpu.VMEM((C,), jnp.int32),
        pltpu.VMEM((C,), jnp.int32),
        pltpu.VMEM((C, D), jnp.float32),
        pltpu.VMEM((C, D), jnp.float32),
        pltpu.VMEM_SHARED((N, D), jnp.float32),
        pltpu.SemaphoreType.DMA,
        pltpu.SemaphoreType.DMA,
    ],
)


def _sc_deg_body(dst_hbm, zeros_hbm, ones_hbm,
                 deg_out, dst_v, ones_v, deg_sh):
    cid = lax.axis_index("c")
    sid = lax.axis_index("s")

    for kz in range(-(-NZ_FULL // NS)):
        zc = sid + NS * kz

        @pl.when(zc < NZ_FULL)
        def _():
            pltpu.sync_copy(zeros_hbm, deg_sh.at[pl.ds(zc * C, C)])

    @pl.when(sid == 0)
    def _():
        pltpu.sync_copy(zeros_hbm.at[pl.ds(0, NTAIL)],
                        deg_sh.at[pl.ds(NZ_FULL * C, NTAIL)])

    pltpu.sync_copy(ones_hbm, ones_v)
    plsc.subcore_barrier()

    @pl.loop(0, KMAX)
    def _(k):
        ci = sid + NS * k

        @pl.when(ci < CHUNKS)
        def _():
            cig = cid * CHUNKS + ci
            pltpu.sync_copy(dst_hbm.at[cig], dst_v)
            pltpu.sync_copy(ones_v, deg_sh.at[dst_v], add=True)

    plsc.subcore_barrier()

    for kz in range(-(-NZ_FULL // NS)):
        zc = sid + NS * kz

        @pl.when(zc < NZ_FULL)
        def _():
            pltpu.sync_copy(deg_sh.at[pl.ds(zc * C, C)],
                            deg_out.at[pl.ds(cid * N + zc * C, C)])

    @pl.when(sid == 0)
    def _():
        pltpu.sync_copy(deg_sh.at[pl.ds(NZ_FULL * C, NTAIL)],
                        deg_out.at[pl.ds(cid * N + NZ_FULL * C, NTAIL)])


_sc_deg = pl.kernel(
    _sc_deg_body,
    out_type=jax.ShapeDtypeStruct((NC * N, D), jnp.float32),
    mesh=_MESH,
    scratch_types=[
        pltpu.VMEM((C,), jnp.int32),
        pltpu.VMEM((C, D), jnp.float32),
        pltpu.VMEM_SHARED((N, D), jnp.float32),
    ],
)


def _dot_f32(a, wt):
    a_hi = a.astype(jnp.bfloat16)
    a_lo = (a - a_hi.astype(jnp.float32)).astype(jnp.bfloat16)
    w_hi = wt.astype(jnp.bfloat16)
    w_lo = (wt - w_hi.astype(jnp.float32)).astype(jnp.bfloat16)
    d = jnp.dot(a_hi, w_hi, preferred_element_type=jnp.float32)
    d += jnp.dot(a_hi, w_lo, preferred_element_type=jnp.float32)
    d += jnp.dot(a_lo, w_hi, preferred_element_type=jnp.float32)
    return d


def _mm_body(a_ref, wt_ref, b_ref, o_ref):
    o_ref[...] = _dot_f32(a_ref[...], wt_ref[...]) + b_ref[...]


_BE = 3200


def _edge_matmul(edge_attr, Wt, b):
    return pl.pallas_call(
        _mm_body,
        grid=(E // _BE,),
        in_specs=[
            pl.BlockSpec((_BE, D), lambda i: (i, 0)),
            pl.BlockSpec((D, D), lambda i: (0, 0)),
            pl.BlockSpec((1, D), lambda i: (0, 0)),
        ],
        out_specs=pl.BlockSpec((_BE, D), lambda i: (i, 0)),
        out_shape=jax.ShapeDtypeStruct((E, D), jnp.float32),
    )(edge_attr, Wt, b)


def _combine_body(acc_ref, deg_ref, o_ref):
    p = acc_ref[:N, :] + acc_ref[N:, :]
    d = deg_ref[:N, 0:1] + deg_ref[N:, 0:1]
    recip = 1.0 / jnp.maximum(d, 1.0)
    o_ref[...] = jnp.maximum(p * recip, 0.0)


def _combine(acc, deg):
    return pl.pallas_call(
        _combine_body,
        out_shape=jax.ShapeDtypeStruct((N, D), jnp.float32),
    )(acc, deg)


def _final_body(acc_ref, deg_ref, wt_ref, b_ref, o_ref):
    p = acc_ref[:N, :] + acc_ref[N:, :]
    d = deg_ref[:N, 0:1] + deg_ref[N:, 0:1]
    recip = 1.0 / jnp.maximum(d, 1.0)
    h = jnp.maximum(p * recip, 0.0)
    o_ref[...] = _dot_f32(h, wt_ref[...]) + b_ref[...]


def _final(acc, deg, Wt, b):
    return pl.pallas_call(
        _final_body,
        out_shape=jax.ShapeDtypeStruct((N, D), jnp.float32),
    )(acc, deg, Wt, b)


def kernel(x, edge_index, edge_attr, W1, b1, W2, b2, W3, b3, Wout, bout):
    src = edge_index[0].reshape(E // C, C)
    dst = edge_index[1].reshape(E // C, C)
    zeros = jnp.zeros((C, D), jnp.float32)
    ones = jnp.ones((C, D), jnp.float32)

    w1 = _edge_matmul(edge_attr, W1.T, b1[None, :])
    w2 = _edge_matmul(edge_attr, W2.T, b2[None, :])
    w3 = _edge_matmul(edge_attr, W3.T, b3[None, :])

    deg = _sc_deg(dst, zeros, ones)
    acc1 = _sc_layer(x, w1, src, dst, zeros)
    h1 = _combine(acc1, deg)
    acc2 = _sc_layer(h1, w2, src, dst, zeros)
    h2 = _combine(acc2, deg)
    acc3 = _sc_layer(h2, w3, src, dst, zeros)
    return _final(acc3, deg, Wout.T, bout[None, :])


# speedup vs baseline: 5.1071x; 1.3759x over previous
import dataclasses

import jax
import jax.numpy as jnp
from jax import lax
from jax.experimental import pallas as pl
from jax.experimental.pallas import tpu as pltpu
from jax.experimental.pallas import tpu_sc as plsc

N = 10000
E = 320000
D = 128

NC = 2
NS = 16
L = 16
NW = NC * NS

EPT = E // NW
CF = 64
NCH = EPT // CF
CT = EPT - NCH * CF

C = 128
NZ_FULL = N // C
NTAIL = N - NZ_FULL * C

_MESH = plsc.VectorSubcoreMesh(core_axis_name="c", subcore_axis_name="s")


def _sc_layer_body(h_hbm, w_hbm, src_hbm, dst_hbm, zeros_hbm, acc_out,
                   src_v0, src_v1, dst_v0, dst_v1, src_t, dst_t,
                   w_v0, w_v1, g_v0, g_v1,
                   acc_sh, sw0, sw1, sg0, sg1, si0, si1):
    src_v = (src_v0, src_v1)
    dst_v = (dst_v0, dst_v1)
    w_v = (w_v0, w_v1)
    g_v = (g_v0, g_v1)
    sem_w = (sw0, sw1)
    sem_g = (sg0, sg1)
    sem_i = (si0, si1)
    cid = lax.axis_index("c")
    sid = lax.axis_index("s")
    wid = cid * NS + sid
    base = wid * EPT

    for kz in range(-(-NZ_FULL // NS)):
        zc = sid + NS * kz

        @pl.when(zc < NZ_FULL)
        def _():
            pltpu.sync_copy(zeros_hbm, acc_sh.at[pl.ds(zc * C, C)])

    @pl.when(sid == 0)
    def _():
        pltpu.sync_copy(zeros_hbm.at[pl.ds(0, NTAIL)],
                        acc_sh.at[pl.ds(NZ_FULL * C, NTAIL)])

    plsc.subcore_barrier()

    def issue_idx(k, p):
        pltpu.async_copy(src_hbm.at[pl.ds(base + k * CF, CF)], src_v[p],
                         sem_i[p])
        pltpu.async_copy(dst_hbm.at[pl.ds(base + k * CF, CF)], dst_v[p],
                         sem_i[p])

    def wait_idx(p):
        pltpu.make_async_copy(src_hbm.at[pl.ds(0, CF)], src_v[p],
                              sem_i[p]).wait()
        pltpu.make_async_copy(dst_hbm.at[pl.ds(0, CF)], dst_v[p],
                              sem_i[p]).wait()

    def issue_wg(k, p):
        pltpu.async_copy(w_hbm.at[pl.ds(base + k * CF, CF)], w_v[p],
                         sem_w[p])
        pltpu.async_copy(h_hbm.at[src_v[p]], g_v[p], sem_g[p])

    def wait_wg(p):
        pltpu.make_async_copy(w_hbm.at[pl.ds(0, CF)], w_v[p],
                              sem_w[p]).wait()
        pltpu.make_async_copy(h_hbm.at[src_v[p]], g_v[p], sem_g[p]).wait()

    issue_idx(0, 0)
    issue_idx(1, 1)
    wait_idx(0)
    issue_wg(0, 0)

    @pl.loop(0, NCH // 2)
    def _(kd):
        for p in range(2):
            k = 2 * kd + p
            q = 1 - p

            @pl.when(k + 1 < NCH)
            def _():
                wait_idx(q)
                issue_wg(k + 1, q)

            wait_wg(p)

            @pl.loop(0, CF)
            def _(r):
                for cc in range(D // L):
                    sl = pl.ds(cc * L, L)
                    g_v[p][r, sl] = g_v[p][r, sl] * w_v[p][r, sl]

            pltpu.sync_copy(g_v[p], acc_sh.at[dst_v[p]], add=True)

            @pl.when(k + 2 < NCH)
            def _():
                issue_idx(k + 2, p)

    pltpu.sync_copy(src_hbm.at[pl.ds(base + NCH * CF, CT)], src_t)
    pltpu.sync_copy(dst_hbm.at[pl.ds(base + NCH * CF, CT)], dst_t)
    pltpu.sync_copy(w_hbm.at[pl.ds(base + NCH * CF, CT)],
                    w_v0.at[pl.ds(0, CT)])
    pltpu.async_copy(h_hbm.at[src_t], g_v0.at[pl.ds(0, CT)], sg0).wait()

    @pl.loop(0, CT)
    def _(r):
        for cc in range(D // L):
            sl = pl.ds(cc * L, L)
            g_v0[r, sl] = g_v0[r, sl] * w_v0[r, sl]

    pltpu.sync_copy(g_v0.at[pl.ds(0, CT)], acc_sh.at[dst_t], add=True)

    plsc.subcore_barrier()

    for kz in range(-(-NZ_FULL // NS)):
        zc = sid + NS * kz

        @pl.when(zc < NZ_FULL)
        def _():
            pltpu.sync_copy(acc_sh.at[pl.ds(zc * C, C)],
                            acc_out.at[pl.ds(cid * N + zc * C, C)])

    @pl.when(sid == 0)
    def _():
        pltpu.sync_copy(acc_sh.at[pl.ds(NZ_FULL * C, NTAIL)],
                        acc_out.at[pl.ds(cid * N + NZ_FULL * C, NTAIL)])


_sc_layer = pl.kernel(
    _sc_layer_body,
    out_type=jax.ShapeDtypeStruct((NC * N, D), jnp.float32),
    mesh=_MESH,
    scratch_types=[
        pltpu.VMEM((CF,), jnp.int32),
        pltpu.VMEM((CF,), jnp.int32),
        pltpu.VMEM((CF,), jnp.int32),
        pltpu.VMEM((CF,), jnp.int32),
        pltpu.VMEM((CT,), jnp.int32),
        pltpu.VMEM((CT,), jnp.int32),
        pltpu.VMEM((CF, D), jnp.float32),
        pltpu.VMEM((CF, D), jnp.float32),
        pltpu.VMEM((CF, D), jnp.float32),
        pltpu.VMEM((CF, D), jnp.float32),
        pltpu.VMEM_SHARED((N, D), jnp.float32),
        pltpu.SemaphoreType.DMA,
        pltpu.SemaphoreType.DMA,
        pltpu.SemaphoreType.DMA,
        pltpu.SemaphoreType.DMA,
        pltpu.SemaphoreType.DMA,
        pltpu.SemaphoreType.DMA,
    ],
)


def _sc_deg_body(dst_hbm, deg_out, dst_slab, deg_local, sem):
    cid = lax.axis_index("c")
    sid = lax.axis_index("s")
    wid = cid * NS + sid

    pltpu.async_copy(dst_hbm.at[pl.ds(wid * EPT, EPT)], dst_slab, sem)

    zeros16 = jnp.zeros((L,), jnp.float32)

    @pl.loop(0, N // L)
    def _(j):
        deg_local[pl.ds(j * L, L)] = zeros16

    pltpu.make_async_copy(dst_hbm.at[pl.ds(0, EPT)], dst_slab, sem).wait()

    ones16 = jnp.ones((L,), jnp.float32)

    @pl.loop(0, EPT // L)
    def _(j):
        idx = dst_slab[pl.ds(j * L, L)]
        plsc.addupdate_scatter(deg_local, [idx], ones16)

    pltpu.sync_copy(deg_local, deg_out.at[wid])


_deg_cp = pltpu.CompilerParams()
if "needs_layout_passes" in pltpu.CompilerParams.__dataclass_fields__:
    _deg_cp = dataclasses.replace(_deg_cp, needs_layout_passes=False)

_sc_deg = pl.kernel(
    _sc_deg_body,
    out_type=jax.ShapeDtypeStruct((NW, N), jnp.float32),
    mesh=_MESH,
    scratch_types=[
        pltpu.VMEM((EPT,), jnp.int32),
        pltpu.VMEM((N,), jnp.float32),
        pltpu.SemaphoreType.DMA,
    ],
    compiler_params=_deg_cp,
)


def _dot_f32(a, wt):
    a_hi = a.astype(jnp.bfloat16)
    a_lo = (a - a_hi.astype(jnp.float32)).astype(jnp.bfloat16)
    w_hi = wt.astype(jnp.bfloat16)
    w_lo = (wt - w_hi.astype(jnp.float32)).astype(jnp.bfloat16)
    d = jnp.dot(a_hi, w_hi, preferred_element_type=jnp.float32)
    d += jnp.dot(a_hi, w_lo, preferred_element_type=jnp.float32)
    d += jnp.dot(a_lo, w_hi, preferred_element_type=jnp.float32)
    return d


def _mm_body(a_ref, wt_ref, b_ref, o_ref):
    o_ref[...] = _dot_f32(a_ref[...], wt_ref[...]) + b_ref[...]


_BE = 3200


def _edge_matmul(edge_attr, Wt, b):
    return pl.pallas_call(
        _mm_body,
        grid=(E // _BE,),
        in_specs=[
            pl.BlockSpec((_BE, D), lambda i: (i, 0)),
            pl.BlockSpec((D, D), lambda i: (0, 0)),
            pl.BlockSpec((1, D), lambda i: (0, 0)),
        ],
        out_specs=pl.BlockSpec((_BE, D), lambda i: (i, 0)),
        out_shape=jax.ShapeDtypeStruct((E, D), jnp.float32),
    )(edge_attr, Wt, b)


def _combine_body(acc_ref, deg_ref, o_ref):
    p = acc_ref[:N, :] + acc_ref[N:, :]
    d = jnp.sum(deg_ref[...], axis=0)[:, None]
    recip = 1.0 / jnp.maximum(d, 1.0)
    o_ref[...] = jnp.maximum(p * recip, 0.0)


def _combine(acc, deg):
    return pl.pallas_call(
        _combine_body,
        out_shape=jax.ShapeDtypeStruct((N, D), jnp.float32),
    )(acc, deg)


def _final_body(acc_ref, deg_ref, wt_ref, b_ref, o_ref):
    p = acc_ref[:N, :] + acc_ref[N:, :]
    d = jnp.sum(deg_ref[...], axis=0)[:, None]
    recip = 1.0 / jnp.maximum(d, 1.0)
    h = jnp.maximum(p * recip, 0.0)
    o_ref[...] = _dot_f32(h, wt_ref[...]) + b_ref[...]


def _final(acc, deg, Wt, b):
    return pl.pallas_call(
        _final_body,
        out_shape=jax.ShapeDtypeStruct((N, D), jnp.float32),
    )(acc, deg, Wt, b)


def kernel(x, edge_index, edge_attr, W1, b1, W2, b2, W3, b3, Wout, bout):
    src = edge_index[0]
    dst = edge_index[1]
    zeros = jnp.zeros((C, D), jnp.float32)

    w1 = _edge_matmul(edge_attr, W1.T, b1[None, :])
    w2 = _edge_matmul(edge_attr, W2.T, b2[None, :])
    w3 = _edge_matmul(edge_attr, W3.T, b3[None, :])

    deg = _sc_deg(dst)
    acc1 = _sc_layer(x, w1, src, dst, zeros)
    h1 = _combine(acc1, deg)
    acc2 = _sc_layer(h1, w2, src, dst, zeros)
    h2 = _combine(acc2, deg)
    acc3 = _sc_layer(h2, w3, src, dst, zeros)
    return _final(acc3, deg, Wout.T, bout[None, :])
